# baseline (speedup 1.0000x reference)
import functools

import jax
import jax.numpy as jnp
from jax import lax
from jax.experimental import pallas as pl
from jax.experimental.pallas import tpu as pltpu
from jax.experimental.pallas import tpu_sc as plsc

_N = 10000
_E = 320000
_H = 128
_G = 128
_C = 10
_NSUB = 16
_NW = 2 * _NSUB
_EPW = _E // _NW
_K = 80
_NCH = _EPW // _K
_RPT = 624
_TAIL = _N - _NSUB * _RPT
_ZR = 48
_BN = 2000


_NBUF = 3


def _seg_agg(y, src, dst):
    mesh = plsc.VectorSubcoreMesh(core_axis_name="c", subcore_axis_name="s")

    @functools.partial(
        pl.kernel,
        mesh=mesh,
        out_type=jax.ShapeDtypeStruct((2, _N, _H), jnp.float32),
        scratch_types=[
            pltpu.VMEM((_EPW,), jnp.int32),
            pltpu.VMEM((_EPW,), jnp.int32),
            pltpu.VMEM((_NBUF, _K, _H), jnp.float32),
            pltpu.VMEM_SHARED((_N, _H), jnp.float32),
            pltpu.SemaphoreType.DMA((_NBUF,)),
            pltpu.SemaphoreType.DMA((_NBUF,)),
            pltpu.SemaphoreType.DMA,
        ],
    )
    def k(y_hbm, src_hbm, dst_hbm, out_hbm, src_all, dst_all, rows, acc, gsem, ssem, zsem):
        c = lax.axis_index("c")
        s = lax.axis_index("s")
        wid = c * _NSUB + s
        base0 = wid * _EPW

        pltpu.sync_copy(src_hbm.at[pl.ds(base0, _EPW)], src_all)
        pltpu.sync_copy(dst_hbm.at[pl.ds(base0, _EPW)], dst_all)

        def fire(i, b):
            pltpu.async_copy(y_hbm.at[src_all.at[pl.ds(i * _K, _K)]], rows.at[b], gsem.at[b])

        fire(0, 0)
        fire(1, 1)

        def zrow(i, carry):
            for j in range(_H // 16):
                rows[2, i, pl.ds(j * 16, 16)] = jnp.zeros((16,), jnp.float32)
            return carry

        lax.fori_loop(0, _K, zrow, 0)
        for r in range(_RPT // _K):
            pltpu.async_copy(rows.at[2], acc.at[pl.ds(s * _RPT + r * _K, _K)], zsem)
        pltpu.async_copy(
            rows.at[2].at[pl.ds(0, _RPT % _K)],
            acc.at[pl.ds(s * _RPT + (_RPT // _K) * _K, _RPT % _K)], zsem)

        @pl.when(s == 0)
        def _zero_tail():
            pltpu.async_copy(rows.at[2].at[pl.ds(0, _TAIL)],
                             acc.at[pl.ds(_NSUB * _RPT, _TAIL)], zsem)

        for r in range(_RPT // _K):
            pltpu.make_async_copy(rows.at[2], acc.at[pl.ds(0, _K)], zsem).wait()
        pltpu.make_async_copy(
            rows.at[2].at[pl.ds(0, _RPT % _K)], acc.at[pl.ds(0, _RPT % _K)], zsem).wait()

        @pl.when(s == 0)
        def _zero_tail_wait():
            pltpu.make_async_copy(
                rows.at[2].at[pl.ds(0, _TAIL)], acc.at[pl.ds(0, _TAIL)], zsem).wait()

        fire(2, 2)
        plsc.subcore_barrier()

        def scat(i, b):
            pltpu.async_copy(rows.at[b], acc.at[dst_all.at[pl.ds(i * _K, _K)]],
                             ssem.at[b], add=True)

        def wait_scat(i, b):
            pltpu.make_async_copy(rows.at[b], acc.at[dst_all.at[pl.ds(i * _K, _K)]],
                                  ssem.at[b]).wait()

        def consume(i, b, do_wait, do_fire):
            pltpu.make_async_copy(y_hbm.at[pl.ds(0, _K)], rows.at[b], gsem.at[b]).wait()
            bp = (b + 2) % _NBUF
            if do_wait:
                wait_scat(i - 1, bp)
            scat(i, b)
            if do_fire:
                fire(i + 2, bp)

        consume(0, 0, False, False)
        consume(1, 1, True, True)

        def group(g, carry):
            i = 2 + g * 3
            consume(i, 2, True, True)
            consume(i + 1, 0, True, True)
            consume(i + 2, 1, True, True)
            return carry

        lax.fori_loop(0, (_NCH - 5) // 3, group, 0)
        consume(_NCH - 3, 2, True, True)
        consume(_NCH - 2, 0, True, False)
        consume(_NCH - 1, 1, True, False)
        wait_scat(_NCH - 1, 1)
        plsc.subcore_barrier()
        pltpu.sync_copy(
            acc.at[pl.ds(s * _RPT, _RPT)],
            out_hbm.at[c, pl.ds(s * _RPT, _RPT)],
        )

        @pl.when(s == 0)
        def _copy_tail():
            pltpu.sync_copy(
                acc.at[pl.ds(_NSUB * _RPT, _TAIL)],
                out_hbm.at[c, pl.ds(_NSUB * _RPT, _TAIL)],
            )

    return k(y, src, dst)


def _lin_in(x, W):
    def body(x_ref, w_ref, o_ref):
        o_ref[...] = jnp.dot(x_ref[...], w_ref[...], preferred_element_type=jnp.float32)

    return pl.pallas_call(
        body,
        grid=(_N // _BN,),
        in_specs=[
            pl.BlockSpec((_BN, _H), lambda i: (i, 0)),
            pl.BlockSpec((_H, _H), lambda i: (0, 0)),
        ],
        out_specs=pl.BlockSpec((_BN, _H), lambda i: (i, 0)),
        out_shape=jax.ShapeDtypeStruct((_N, _H), jnp.float32),
    )(x, W)


def _lin_mid(p, b, W):
    def body(p_ref, b_ref, w_ref, o_ref):
        h = jnp.maximum(p_ref[0] + p_ref[1] + b_ref[...], 0.0)
        o_ref[...] = jnp.dot(h, w_ref[...], preferred_element_type=jnp.float32)

    return pl.pallas_call(
        body,
        grid=(_N // _BN,),
        in_specs=[
            pl.BlockSpec((2, _BN, _H), lambda i: (0, i, 0)),
            pl.BlockSpec((1, _H), lambda i: (0, 0)),
            pl.BlockSpec((_H, _H), lambda i: (0, 0)),
        ],
        out_specs=pl.BlockSpec((_BN, _H), lambda i: (i, 0)),
        out_shape=jax.ShapeDtypeStruct((_N, _H), jnp.float32),
    )(p, b, W)


def _final(p, b3, gid3, Wc1, bc1, Wc2, bc2, Wc3, bc3):
    def body(gid_ref, p_ref, b_ref, wc1, v1, wc2, v2, wc3, v3, o_ref, sums, counts):
        i = pl.program_id(0)

        @pl.when(i == 0)
        def _init():
            sums[...] = jnp.zeros_like(sums)
            counts[...] = jnp.zeros_like(counts)

        h = jnp.maximum(p_ref[0] + p_ref[1] + b_ref[...], 0.0)
        gid = gid_ref[0]
        mask = (lax.broadcasted_iota(jnp.int32, (_G, _BN), 0) == gid).astype(jnp.float32)
        sums[...] += jnp.dot(mask, h, preferred_element_type=jnp.float32)
        counts[...] += jnp.sum(mask, axis=1, keepdims=True)

        @pl.when(i == pl.num_programs(0) - 1)
        def _fin():
            hg = sums[...] / jnp.maximum(counts[...], 1.0)
            z = jnp.maximum(jnp.dot(hg, wc1[...], preferred_element_type=jnp.float32) + v1[...], 0.0)
            z = jnp.maximum(jnp.dot(z, wc2[...], preferred_element_type=jnp.float32) + v2[...], 0.0)
            o_ref[...] = jnp.dot(z, wc3[...], preferred_element_type=jnp.float32) + v3[...]

    return pl.pallas_call(
        body,
        grid=(_N // _BN,),
        in_specs=[
            pl.BlockSpec((1, 1, _BN), lambda i: (i, 0, 0)),
            pl.BlockSpec((2, _BN, _H), lambda i: (0, i, 0)),
            pl.BlockSpec((1, _H), lambda i: (0, 0)),
            pl.BlockSpec((_H, _H), lambda i: (0, 0)),
            pl.BlockSpec((1, _H), lambda i: (0, 0)),
            pl.BlockSpec((_H, _H), lambda i: (0, 0)),
            pl.BlockSpec((1, _H), lambda i: (0, 0)),
            pl.BlockSpec((_H, _C), lambda i: (0, 0)),
            pl.BlockSpec((1, _C), lambda i: (0, 0)),
        ],
        out_specs=pl.BlockSpec((_G, _C), lambda i: (0, 0)),
        out_shape=jax.ShapeDtypeStruct((_G, _C), jnp.float32),
        scratch_shapes=[
            pltpu.VMEM((_G, _H), jnp.float32),
            pltpu.VMEM((_G, 1), jnp.float32),
        ],
    )(gid3, p, b3, Wc1, bc1, Wc2, bc2, Wc3, bc3)


def kernel(x, edge_index, graph_ids, W1, b1, W2, b2, W3, b3, Wc1, bc1, Wc2, bc2, Wc3, bc3):
    src = edge_index[0]
    dst = edge_index[1]
    gid3 = graph_ids.reshape(_N // _BN, 1, _BN)

    y1 = _lin_in(x, W1)
    p1 = _seg_agg(y1, src, dst)
    y2 = _lin_mid(p1, b1.reshape(1, _H), W2)
    p2 = _seg_agg(y2, src, dst)
    y3 = _lin_mid(p2, b2.reshape(1, _H), W3)
    p3 = _seg_agg(y3, src, dst)
    return _final(
        p3, b3.reshape(1, _H), gid3,
        Wc1, bc1.reshape(1, _H), Wc2, bc2.reshape(1, _H), Wc3, bc3.reshape(1, _C),
    )


# speedup vs baseline: 1.0366x; 1.0366x over previous
import functools

import jax
import jax.numpy as jnp
from jax import lax
from jax.experimental import pallas as pl
from jax.experimental.pallas import tpu as pltpu
from jax.experimental.pallas import tpu_sc as plsc

_N = 10000
_E = 320000
_H = 128
_G = 128
_C = 10
_NSUB = 16
_NW = 2 * _NSUB
_EPW = _E // _NW
_K = 80
_NCH = _EPW // _K
_RPT = 624
_TAIL = _N - _NSUB * _RPT
_ZR = 48
_BN = 2000


_NBUF = 3


def _seg_agg(y, src, dst):
    mesh = plsc.VectorSubcoreMesh(core_axis_name="c", subcore_axis_name="s")

    @functools.partial(
        pl.kernel,
        mesh=mesh,
        out_type=jax.ShapeDtypeStruct((2, _N, _H), jnp.float32),
        scratch_types=[
            pltpu.VMEM((_EPW,), jnp.int32),
            pltpu.VMEM((_NBUF, _K), jnp.int32),
            pltpu.VMEM((_NBUF, _K, _H), jnp.float32),
            pltpu.VMEM((_ZR, _H), jnp.float32),
            pltpu.VMEM_SHARED((_N, _H), jnp.float32),
            pltpu.SemaphoreType.DMA((_NBUF,)),
            pltpu.SemaphoreType.DMA((_NBUF,)),
        ],
    )
    def k(y_hbm, src_hbm, dst_hbm, out_hbm, src_all, dstv, rows, zbuf, acc, gsem, dsem):
        c = lax.axis_index("c")
        s = lax.axis_index("s")
        wid = c * _NSUB + s
        base0 = wid * _EPW

        pltpu.sync_copy(src_hbm.at[pl.ds(base0, _EPW)], src_all)

        def fire(i, b):
            pltpu.async_copy(dst_hbm.at[pl.ds(base0 + i * _K, _K)], dstv.at[b], dsem.at[b])
            pltpu.async_copy(y_hbm.at[src_all.at[pl.ds(i * _K, _K)]], rows.at[b], gsem.at[b])

        for b in range(_NBUF):
            fire(b, b)

        def zrow(i, carry):
            for j in range(_H // 16):
                zbuf[i, pl.ds(j * 16, 16)] = jnp.zeros((16,), jnp.float32)
            return carry

        lax.fori_loop(0, _ZR, zrow, 0)
        for r in range(_RPT // _ZR):
            pltpu.sync_copy(zbuf, acc.at[pl.ds(s * _RPT + r * _ZR, _ZR)])

        @pl.when(s == 0)
        def _zero_tail():
            pltpu.sync_copy(zbuf.at[pl.ds(0, _TAIL)], acc.at[pl.ds(_NSUB * _RPT, _TAIL)])

        plsc.subcore_barrier()

        def consume(i, b, may_prefetch):
            pltpu.make_async_copy(dst_hbm.at[pl.ds(0, _K)], dstv.at[b], dsem.at[b]).wait()
            pltpu.make_async_copy(y_hbm.at[pl.ds(0, _K)], rows.at[b], gsem.at[b]).wait()
            pltpu.sync_copy(rows.at[b], acc.at[dstv.at[b]], add=True)
            if may_prefetch:
                @pl.when(i + _NBUF < _NCH)
                def _prefetch():
                    fire(i + _NBUF, b)

        def group(g, carry):
            for b in range(_NBUF):
                consume(g * _NBUF + b, b, True)
            return carry

        lax.fori_loop(0, _NCH // _NBUF, group, 0)
        for i in range((_NCH // _NBUF) * _NBUF, _NCH):
            consume(i, i % _NBUF, False)
        plsc.subcore_barrier()
        pltpu.sync_copy(
            acc.at[pl.ds(s * _RPT, _RPT)],
            out_hbm.at[c, pl.ds(s * _RPT, _RPT)],
        )

        @pl.when(s == 0)
        def _copy_tail():
            pltpu.sync_copy(
                acc.at[pl.ds(_NSUB * _RPT, _TAIL)],
                out_hbm.at[c, pl.ds(_NSUB * _RPT, _TAIL)],
            )

    return k(y, src, dst)


def _lin_in(x, W):
    def body(x_ref, w_ref, o_ref):
        o_ref[...] = jnp.dot(x_ref[...], w_ref[...], preferred_element_type=jnp.float32)

    return pl.pallas_call(
        body,
        grid=(_N // _BN,),
        in_specs=[
            pl.BlockSpec((_BN, _H), lambda i: (i, 0)),
            pl.BlockSpec((_H, _H), lambda i: (0, 0)),
        ],
        out_specs=pl.BlockSpec((_BN, _H), lambda i: (i, 0)),
        out_shape=jax.ShapeDtypeStruct((_N, _H), jnp.float32),
    )(x, W)


def _lin_mid(p, b, W):
    def body(p_ref, b_ref, w_ref, o_ref):
        h = jnp.maximum(p_ref[0] + p_ref[1] + b_ref[...], 0.0)
        o_ref[...] = jnp.dot(h, w_ref[...], preferred_element_type=jnp.float32)

    return pl.pallas_call(
        body,
        grid=(_N // _BN,),
        in_specs=[
            pl.BlockSpec((2, _BN, _H), lambda i: (0, i, 0)),
            pl.BlockSpec((1, _H), lambda i: (0, 0)),
            pl.BlockSpec((_H, _H), lambda i: (0, 0)),
        ],
        out_specs=pl.BlockSpec((_BN, _H), lambda i: (i, 0)),
        out_shape=jax.ShapeDtypeStruct((_N, _H), jnp.float32),
    )(p, b, W)


def _final(p, b3, gid3, Wc1, bc1, Wc2, bc2, Wc3, bc3):
    def body(gid_ref, p_ref, b_ref, wc1, v1, wc2, v2, wc3, v3, o_ref, sums, counts):
        i = pl.program_id(0)

        @pl.when(i == 0)
        def _init():
            sums[...] = jnp.zeros_like(sums)
            counts[...] = jnp.zeros_like(counts)

        h = jnp.maximum(p_ref[0] + p_ref[1] + b_ref[...], 0.0)
        gid = gid_ref[0]
        mask = (lax.broadcasted_iota(jnp.int32, (_G, _BN), 0) == gid).astype(jnp.float32)
        sums[...] += jnp.dot(mask, h, preferred_element_type=jnp.float32)
        counts[...] += jnp.sum(mask, axis=1, keepdims=True)

        @pl.when(i == pl.num_programs(0) - 1)
        def _fin():
            hg = sums[...] / jnp.maximum(counts[...], 1.0)
            z = jnp.maximum(jnp.dot(hg, wc1[...], preferred_element_type=jnp.float32) + v1[...], 0.0)
            z = jnp.maximum(jnp.dot(z, wc2[...], preferred_element_type=jnp.float32) + v2[...], 0.0)
            o_ref[...] = jnp.dot(z, wc3[...], preferred_element_type=jnp.float32) + v3[...]

    return pl.pallas_call(
        body,
        grid=(_N // _BN,),
        in_specs=[
            pl.BlockSpec((1, 1, _BN), lambda i: (i, 0, 0)),
            pl.BlockSpec((2, _BN, _H), lambda i: (0, i, 0)),
            pl.BlockSpec((1, _H), lambda i: (0, 0)),
            pl.BlockSpec((_H, _H), lambda i: (0, 0)),
            pl.BlockSpec((1, _H), lambda i: (0, 0)),
            pl.BlockSpec((_H, _H), lambda i: (0, 0)),
            pl.BlockSpec((1, _H), lambda i: (0, 0)),
            pl.BlockSpec((_H, _C), lambda i: (0, 0)),
            pl.BlockSpec((1, _C), lambda i: (0, 0)),
        ],
        out_specs=pl.BlockSpec((_G, _C), lambda i: (0, 0)),
        out_shape=jax.ShapeDtypeStruct((_G, _C), jnp.float32),
        scratch_shapes=[
            pltpu.VMEM((_G, _H), jnp.float32),
            pltpu.VMEM((_G, 1), jnp.float32),
        ],
    )(gid3, p, b3, Wc1, bc1, Wc2, bc2, Wc3, bc3)


def kernel(x, edge_index, graph_ids, W1, b1, W2, b2, W3, b3, Wc1, bc1, Wc2, bc2, Wc3, bc3):
    src = edge_index[0]
    dst = edge_index[1]
    gid3 = graph_ids.reshape(_N // _BN, 1, _BN)

    y1 = _lin_in(x, W1)
    p1 = _seg_agg(y1, src, dst)
    y2 = _lin_mid(p1, b1.reshape(1, _H), W2)
    p2 = _seg_agg(y2, src, dst)
    y3 = _lin_mid(p2, b2.reshape(1, _H), W3)
    p3 = _seg_agg(y3, src, dst)
    return _final(
        p3, b3.reshape(1, _H), gid3,
        Wc1, bc1.reshape(1, _H), Wc2, bc2.reshape(1, _H), Wc3, bc3.reshape(1, _C),
    )
